# baseline (speedup 1.0000x reference)
import functools
import math

import jax
import jax.numpy as jnp
import numpy as np
from jax import lax
from jax.experimental import pallas as pl
from jax.experimental.pallas import tpu as pltpu
from jax.experimental.pallas import tpu_sc as plsc

D_MODEL = 64
VOCAB = 1000000
BATCH = 1024
SEQ = 200

NC = 2
NS = 16
NW = NC * NS

B_PER_W = BATCH // NW


def _make_pe():
    position = np.arange(0, SEQ, dtype=np.float32)[:, None]
    div_term = np.exp(
        np.arange(0, D_MODEL, 2, dtype=np.float32) * -(math.log(10000.0) / D_MODEL)
    )
    pe = np.zeros((SEQ, D_MODEL), dtype=np.float32)
    pe[:, 0::2] = np.sin(position * div_term)
    pe[:, 1::2] = np.cos(position * div_term)
    return pe


_PE = _make_pe()
_SCALE = math.sqrt(D_MODEL)


RBLK = 256
NFULL = 3906
RTAIL = VOCAB - NFULL * RBLK
RLOOP = (NFULL + NW - 1) // NW


@functools.partial(
    pl.kernel,
    mesh=plsc.VectorSubcoreMesh(core_axis_name="c", subcore_axis_name="s"),
    out_type=jax.ShapeDtypeStruct((VOCAB, D_MODEL), jnp.float32),
    compiler_params=pltpu.CompilerParams(
        use_tc_tiling_on_sc=True, needs_layout_passes=False),
    scratch_types=[
        pltpu.VMEM((D_MODEL, RBLK), jnp.float32),
        pltpu.VMEM((D_MODEL, RBLK), jnp.float32),
        pltpu.VMEM((RBLK, D_MODEL), jnp.float32),
        pltpu.VMEM((RBLK, D_MODEL), jnp.float32),
        pltpu.SemaphoreType.DMA,
        pltpu.SemaphoreType.DMA,
        pltpu.SemaphoreType.DMA,
        pltpu.SemaphoreType.DMA,
    ],
)
def _sc_repack(t2_hbm, tail_hbm, scratch_hbm,
               in0, in1, tr0, tr1, i0, i1, o0, o1):
    wid = lax.axis_index("s") * NC + lax.axis_index("c")
    inb = (in0, in1)
    trb = (tr0, tr1)
    isem = (i0, i1)
    osem = (o0, o1)
    iota16 = lax.iota(jnp.int32, 16)

    def v0_of(n):
        return (wid + NW * n) * RBLK

    def start_in(n, p):
        pltpu.async_copy(
            t2_hbm.at[:, pl.ds(v0_of(n), RBLK)], inb[p], isem[p])

    def wait_in(p):
        pltpu.make_async_copy(
            t2_hbm.at[:, pl.ds(0, RBLK)], inb[p], isem[p]).wait()

    def wait_out(p):
        pltpu.make_async_copy(
            trb[p], scratch_hbm.at[pl.ds(0, RBLK)], osem[p]).wait()

    start_in(0, 0)

    def body(n, carry):
        g = wid + NW * n

        def for_parity(b, nb):
            @pl.when(g + NW < NFULL)
            def _():
                start_in(n + 1, nb)

            wait_in(b)

            @pl.when(n >= 2)
            def _():
                wait_out(b)

            def row(v, carry2):
                bvec = iota16 * 0 + v
                for d0 in range(0, D_MODEL, 16):
                    vec = plsc.load_gather(inb[b], [d0 + iota16, bvec])
                    trb[b][v, pl.ds(d0, 16)] = vec
                return carry2

            lax.fori_loop(0, RBLK, row, 0, unroll=4)
            pltpu.async_copy(
                trb[b], scratch_hbm.at[pl.ds(v0_of(n), RBLK)], osem[b])

        @pl.when(g < NFULL)
        def _():
            @pl.when(n % 2 == 0)
            def _():
                for_parity(0, 1)

            @pl.when(n % 2 == 1)
            def _():
                for_parity(1, 0)

        return carry

    lax.fori_loop(0, RLOOP, body, 0)
    wait_out(0)
    wait_out(1)

    @pl.when(wid == NW - 1)
    def _():
        pltpu.sync_copy(tail_hbm, scratch_hbm.at[pl.ds(NFULL * RBLK, RTAIL)])


NGRP = SEQ // 16


@functools.partial(
    pl.kernel,
    mesh=plsc.VectorSubcoreMesh(core_axis_name="c", subcore_axis_name="s"),
    out_type=jax.ShapeDtypeStruct((BATCH * SEQ, D_MODEL), jnp.float32),
    compiler_params=pltpu.CompilerParams(
        use_tc_tiling_on_sc=True, needs_layout_passes=False),
    scratch_types=[
        pltpu.VMEM((SEQ, 128), jnp.int32),
        pltpu.VMEM((SEQ, D_MODEL), jnp.float32),
        pltpu.VMEM((SEQ, D_MODEL), jnp.float32),
        pltpu.VMEM((SEQ, D_MODEL), jnp.float32),
        pltpu.SemaphoreType.DMA,
        pltpu.SemaphoreType.DMA,
        pltpu.SemaphoreType.DMA,
        pltpu.SemaphoreType.DMA,
    ],
)
def _sc_gather(xt_hbm, pe_hbm, table_hbm, out_hbm,
               idx_v, pe_v, rows0, rows1, g0, g1, o0, o1):
    wid = lax.axis_index("s") * NC + lax.axis_index("c")
    rows = (rows0, rows1)
    gsem = (g0, g1)
    osem = (o0, o1)

    blk = (wid // 4) * 128
    coff = (wid % 4) * B_PER_W
    pltpu.sync_copy(xt_hbm.at[:, pl.ds(blk, 128)], idx_v)
    pltpu.sync_copy(pe_hbm, pe_v)

    iota16 = lax.iota(jnp.int32, 16)

    def enqueue_gathers(j, p):
        bcol = iota16 * 0 + (coff + j)

        def grp(g, carry):
            svec = g * 16 + iota16
            v = plsc.load_gather(idx_v, [svec, bcol])
            base = g * 16
            for t in range(16):
                pltpu.async_copy(
                    table_hbm.at[v[t]], rows[p].at[base + t], gsem[p])
            return carry

        lax.fori_loop(0, NGRP, grp, 0)
        svec = (SEQ - 16) + iota16
        v = plsc.load_gather(idx_v, [svec, bcol])
        for t in range(8, 16):
            pltpu.async_copy(
                table_hbm.at[v[t]], rows[p].at[SEQ - 16 + t], gsem[p])

    def wait_gathers(p):
        pltpu.make_async_copy(
            table_hbm.at[pl.ds(0, SEQ)], rows[p], gsem[p]).wait()

    def compute(p):
        rbuf = rows[p]

        def row(j, carry):
            for db in range(D_MODEL // 16):
                sl = pl.ds(db * 16, 16)
                rbuf[j, sl] = rbuf[j, sl] * _SCALE + pe_v[j, sl]
            return carry

        lax.fori_loop(0, SEQ, row, 0, unroll=2)

    def start_out(j, p):
        pltpu.async_copy(
            rows[p], out_hbm.at[pl.ds((wid * B_PER_W + j) * SEQ, SEQ)], osem[p])

    def wait_out(p):
        pltpu.make_async_copy(
            rows[p], out_hbm.at[pl.ds(0, SEQ)], osem[p]).wait()

    enqueue_gathers(0, 0)

    def step(k, carry):
        def for_parity(b, nb):
            @pl.when(k >= 1)
            def _():
                wait_out(nb)

            @pl.when(k + 1 < B_PER_W)
            def _():
                enqueue_gathers(k + 1, nb)

            wait_gathers(b)
            compute(b)
            start_out(k, b)

        @pl.when(k % 2 == 0)
        def _():
            for_parity(0, 1)

        @pl.when(k % 2 == 1)
        def _():
            for_parity(1, 0)

        return carry

    lax.fori_loop(0, B_PER_W, step, 0)
    wait_out((B_PER_W - 1) % 2)


def kernel(x, emb_table):
    scratch = _sc_repack(emb_table.T, emb_table[NFULL * RBLK:])
    out = _sc_gather(x.T, jnp.asarray(_PE), scratch)
    return out.reshape(BATCH, SEQ, D_MODEL)


# speedup vs baseline: 1.0017x; 1.0017x over previous
import functools
import math

import jax
import jax.numpy as jnp
import numpy as np
from jax import lax
from jax.experimental import pallas as pl
from jax.experimental.pallas import tpu as pltpu
from jax.experimental.pallas import tpu_sc as plsc

D_MODEL = 64
VOCAB = 1000000
BATCH = 1024
SEQ = 200

NC = 2
NS = 16
NW = NC * NS

B_PER_W = BATCH // NW


def _make_pe():
    position = np.arange(0, SEQ, dtype=np.float32)[:, None]
    div_term = np.exp(
        np.arange(0, D_MODEL, 2, dtype=np.float32) * -(math.log(10000.0) / D_MODEL)
    )
    pe = np.zeros((SEQ, D_MODEL), dtype=np.float32)
    pe[:, 0::2] = np.sin(position * div_term)
    pe[:, 1::2] = np.cos(position * div_term)
    return pe


_PE = _make_pe()
_SCALE = math.sqrt(D_MODEL)


RBLK = 256
NFULL = 3906
RTAIL = VOCAB - NFULL * RBLK
RLOOP = (NFULL + NW - 1) // NW


@functools.partial(
    pl.kernel,
    mesh=plsc.VectorSubcoreMesh(core_axis_name="c", subcore_axis_name="s"),
    out_type=jax.ShapeDtypeStruct((VOCAB, D_MODEL), jnp.float32),
    compiler_params=pltpu.CompilerParams(
        use_tc_tiling_on_sc=True, needs_layout_passes=False),
    scratch_types=[
        pltpu.VMEM((D_MODEL, RBLK + 1), jnp.float32),
        pltpu.VMEM((D_MODEL, RBLK + 1), jnp.float32),
        pltpu.VMEM((RBLK, D_MODEL), jnp.float32),
        pltpu.VMEM((RBLK, D_MODEL), jnp.float32),
        pltpu.SemaphoreType.DMA,
        pltpu.SemaphoreType.DMA,
        pltpu.SemaphoreType.DMA,
        pltpu.SemaphoreType.DMA,
    ],
)
def _sc_repack(t2_hbm, tail_hbm, scratch_hbm,
               in0, in1, tr0, tr1, i0, i1, o0, o1):
    wid = lax.axis_index("s") * NC + lax.axis_index("c")
    inb = (in0, in1)
    trb = (tr0, tr1)
    isem = (i0, i1)
    osem = (o0, o1)
    iota16 = lax.iota(jnp.int32, 16)

    def v0_of(n):
        return (wid + NW * n) * RBLK

    def start_in(n, p):
        pltpu.async_copy(
            t2_hbm.at[:, pl.ds(v0_of(n), RBLK)],
            inb[p].at[:, pl.ds(0, RBLK)], isem[p])

    def wait_in(p):
        pltpu.make_async_copy(
            t2_hbm.at[:, pl.ds(0, RBLK)],
            inb[p].at[:, pl.ds(0, RBLK)], isem[p]).wait()

    def wait_out(p):
        pltpu.make_async_copy(
            trb[p], scratch_hbm.at[pl.ds(0, RBLK)], osem[p]).wait()

    start_in(0, 0)

    def body(n, carry):
        g = wid + NW * n

        def for_parity(b, nb):
            @pl.when(g + NW < NFULL)
            def _():
                start_in(n + 1, nb)

            wait_in(b)

            @pl.when(n >= 2)
            def _():
                wait_out(b)

            def row(v, carry2):
                bvec = iota16 * 0 + v
                for d0 in range(0, D_MODEL, 16):
                    vec = plsc.load_gather(inb[b], [d0 + iota16, bvec])
                    trb[b][v, pl.ds(d0, 16)] = vec
                return carry2

            lax.fori_loop(0, RBLK, row, 0, unroll=4)
            pltpu.async_copy(
                trb[b], scratch_hbm.at[pl.ds(v0_of(n), RBLK)], osem[b])

        @pl.when(g < NFULL)
        def _():
            @pl.when(n % 2 == 0)
            def _():
                for_parity(0, 1)

            @pl.when(n % 2 == 1)
            def _():
                for_parity(1, 0)

        return carry

    lax.fori_loop(0, RLOOP, body, 0)
    wait_out(0)
    wait_out(1)

    @pl.when(wid == NW - 1)
    def _():
        pltpu.sync_copy(tail_hbm, scratch_hbm.at[pl.ds(NFULL * RBLK, RTAIL)])


NGRP = SEQ // 16


@functools.partial(
    pl.kernel,
    mesh=plsc.VectorSubcoreMesh(core_axis_name="c", subcore_axis_name="s"),
    out_type=jax.ShapeDtypeStruct((BATCH * SEQ, D_MODEL), jnp.float32),
    compiler_params=pltpu.CompilerParams(
        use_tc_tiling_on_sc=True, needs_layout_passes=False),
    scratch_types=[
        pltpu.VMEM((SEQ, 128), jnp.int32),
        pltpu.VMEM((SEQ, D_MODEL), jnp.float32),
        pltpu.VMEM((SEQ, D_MODEL), jnp.float32),
        pltpu.VMEM((SEQ, D_MODEL), jnp.float32),
        pltpu.SemaphoreType.DMA,
        pltpu.SemaphoreType.DMA,
        pltpu.SemaphoreType.DMA,
        pltpu.SemaphoreType.DMA,
    ],
)
def _sc_gather(xt_hbm, pe_hbm, table_hbm, out_hbm,
               idx_v, pe_v, rows0, rows1, g0, g1, o0, o1):
    wid = lax.axis_index("s") * NC + lax.axis_index("c")
    rows = (rows0, rows1)
    gsem = (g0, g1)
    osem = (o0, o1)

    blk = (wid // 4) * 128
    coff = (wid % 4) * B_PER_W
    pltpu.sync_copy(xt_hbm.at[:, pl.ds(blk, 128)], idx_v)
    pltpu.sync_copy(pe_hbm, pe_v)

    iota16 = lax.iota(jnp.int32, 16)

    def enqueue_gathers(j, p):
        bcol = iota16 * 0 + (coff + j)

        def grp(g, carry):
            svec = g * 16 + iota16
            v = plsc.load_gather(idx_v, [svec, bcol])
            base = g * 16
            for t in range(16):
                pltpu.async_copy(
                    table_hbm.at[v[t]], rows[p].at[base + t], gsem[p])
            return carry

        lax.fori_loop(0, NGRP, grp, 0)
        svec = (SEQ - 16) + iota16
        v = plsc.load_gather(idx_v, [svec, bcol])
        for t in range(8, 16):
            pltpu.async_copy(
                table_hbm.at[v[t]], rows[p].at[SEQ - 16 + t], gsem[p])

    def wait_gathers(p):
        pltpu.make_async_copy(
            table_hbm.at[pl.ds(0, SEQ)], rows[p], gsem[p]).wait()

    def compute(p):
        rbuf = rows[p]

        def row(j, carry):
            for db in range(D_MODEL // 16):
                sl = pl.ds(db * 16, 16)
                rbuf[j, sl] = rbuf[j, sl] * _SCALE + pe_v[j, sl]
            return carry

        lax.fori_loop(0, SEQ, row, 0, unroll=2)

    def start_out(j, p):
        pltpu.async_copy(
            rows[p], out_hbm.at[pl.ds((wid * B_PER_W + j) * SEQ, SEQ)], osem[p])

    def wait_out(p):
        pltpu.make_async_copy(
            rows[p], out_hbm.at[pl.ds(0, SEQ)], osem[p]).wait()

    enqueue_gathers(0, 0)

    def step(k, carry):
        def for_parity(b, nb):
            @pl.when(k >= 1)
            def _():
                wait_out(nb)

            @pl.when(k + 1 < B_PER_W)
            def _():
                enqueue_gathers(k + 1, nb)

            wait_gathers(b)
            compute(b)
            start_out(k, b)

        @pl.when(k % 2 == 0)
        def _():
            for_parity(0, 1)

        @pl.when(k % 2 == 1)
        def _():
            for_parity(1, 0)

        return carry

    lax.fori_loop(0, B_PER_W, step, 0)
    wait_out((B_PER_W - 1) % 2)


def kernel(x, emb_table):
    scratch = _sc_repack(emb_table.T, emb_table[NFULL * RBLK:])
    out = _sc_gather(x.T, jnp.asarray(_PE), scratch)
    return out.reshape(BATCH, SEQ, D_MODEL)


# speedup vs baseline: 1.6870x; 1.6841x over previous
import functools
import math

import jax
import jax.numpy as jnp
import numpy as np
from jax import lax
from jax.experimental import pallas as pl
from jax.experimental.pallas import tpu as pltpu
from jax.experimental.pallas import tpu_sc as plsc

D_MODEL = 64
VOCAB = 1000000
BATCH = 1024
SEQ = 200

NC = 2
NS = 16
NW = NC * NS

B_PER_W = BATCH // NW


def _make_pe():
    position = np.arange(0, SEQ, dtype=np.float32)[:, None]
    div_term = np.exp(
        np.arange(0, D_MODEL, 2, dtype=np.float32) * -(math.log(10000.0) / D_MODEL)
    )
    pe = np.zeros((SEQ, D_MODEL), dtype=np.float32)
    pe[:, 0::2] = np.sin(position * div_term)
    pe[:, 1::2] = np.cos(position * div_term)
    return pe


_PE = _make_pe()
_SCALE = math.sqrt(D_MODEL)


RBLK = 256
NFULL = 3906
RTAIL = VOCAB - NFULL * RBLK
RLOOP = (NFULL + NW - 1) // NW


@functools.partial(
    pl.kernel,
    mesh=plsc.VectorSubcoreMesh(core_axis_name="c", subcore_axis_name="s"),
    out_type=jax.ShapeDtypeStruct((VOCAB, D_MODEL), jnp.float32),
    compiler_params=pltpu.CompilerParams(
        use_tc_tiling_on_sc=True, needs_layout_passes=False),
    scratch_types=[
        pltpu.VMEM((D_MODEL, RBLK + 1), jnp.float32),
        pltpu.VMEM((D_MODEL, RBLK + 1), jnp.float32),
        pltpu.VMEM((RBLK, D_MODEL), jnp.float32),
        pltpu.VMEM((RBLK, D_MODEL), jnp.float32),
        pltpu.SemaphoreType.DMA,
        pltpu.SemaphoreType.DMA,
        pltpu.SemaphoreType.DMA,
        pltpu.SemaphoreType.DMA,
    ],
)
def _sc_repack(t2_hbm, tail_hbm, scratch_hbm,
               in0, in1, tr0, tr1, i0, i1, o0, o1):
    wid = lax.axis_index("s") * NC + lax.axis_index("c")
    inb = (in0, in1)
    trb = (tr0, tr1)
    isem = (i0, i1)
    osem = (o0, o1)
    iota16 = lax.iota(jnp.int32, 16)

    def v0_of(n):
        return (wid + NW * n) * RBLK

    def start_in(n, p):
        pltpu.async_copy(
            t2_hbm.at[:, pl.ds(v0_of(n), RBLK)],
            inb[p].at[:, pl.ds(0, RBLK)], isem[p])

    def wait_in(p):
        pltpu.make_async_copy(
            t2_hbm.at[:, pl.ds(0, RBLK)],
            inb[p].at[:, pl.ds(0, RBLK)], isem[p]).wait()

    def wait_out(p):
        pltpu.make_async_copy(
            trb[p], scratch_hbm.at[pl.ds(0, RBLK)], osem[p]).wait()

    start_in(0, 0)

    def body(n, carry):
        g = wid + NW * n

        def for_parity(b, nb):
            @pl.when(g + NW < NFULL)
            def _():
                start_in(n + 1, nb)

            wait_in(b)

            @pl.when(n >= 2)
            def _():
                wait_out(b)

            @plsc.parallel_loop(0, RBLK, unroll=4)
            def row(v):
                bvec = iota16 * 0 + v
                for d0 in range(0, D_MODEL, 16):
                    vec = plsc.load_gather(inb[b], [d0 + iota16, bvec])
                    trb[b][v, pl.ds(d0, 16)] = vec
            pltpu.async_copy(
                trb[b], scratch_hbm.at[pl.ds(v0_of(n), RBLK)], osem[b])

        @pl.when(g < NFULL)
        def _():
            @pl.when(n % 2 == 0)
            def _():
                for_parity(0, 1)

            @pl.when(n % 2 == 1)
            def _():
                for_parity(1, 0)

        return carry

    lax.fori_loop(0, RLOOP, body, 0)
    wait_out(0)
    wait_out(1)

    @pl.when(wid == NW - 1)
    def _():
        pltpu.sync_copy(tail_hbm, scratch_hbm.at[pl.ds(NFULL * RBLK, RTAIL)])


NGRP = SEQ // 16


@functools.partial(
    pl.kernel,
    mesh=plsc.VectorSubcoreMesh(core_axis_name="c", subcore_axis_name="s"),
    out_type=jax.ShapeDtypeStruct((BATCH * SEQ, D_MODEL), jnp.float32),
    compiler_params=pltpu.CompilerParams(
        use_tc_tiling_on_sc=True, needs_layout_passes=False),
    scratch_types=[
        pltpu.VMEM((SEQ, 128), jnp.int32),
        pltpu.VMEM((SEQ, D_MODEL), jnp.float32),
        pltpu.VMEM((SEQ, D_MODEL), jnp.float32),
        pltpu.VMEM((SEQ, D_MODEL), jnp.float32),
        pltpu.SemaphoreType.DMA,
        pltpu.SemaphoreType.DMA,
        pltpu.SemaphoreType.DMA,
        pltpu.SemaphoreType.DMA,
    ],
)
def _sc_gather(xt_hbm, pe_hbm, table_hbm, out_hbm,
               idx_v, pe_v, rows0, rows1, g0, g1, o0, o1):
    wid = lax.axis_index("s") * NC + lax.axis_index("c")
    rows = (rows0, rows1)
    gsem = (g0, g1)
    osem = (o0, o1)

    blk = (wid // 4) * 128
    coff = (wid % 4) * B_PER_W
    pltpu.sync_copy(xt_hbm.at[:, pl.ds(blk, 128)], idx_v)
    pltpu.sync_copy(pe_hbm, pe_v)

    iota16 = lax.iota(jnp.int32, 16)

    def enqueue_gathers(j, p):
        bcol = iota16 * 0 + (coff + j)

        def grp(g, carry):
            svec = g * 16 + iota16
            v = plsc.load_gather(idx_v, [svec, bcol])
            base = g * 16
            for t in range(16):
                pltpu.async_copy(
                    table_hbm.at[v[t]], rows[p].at[base + t], gsem[p])
            return carry

        lax.fori_loop(0, NGRP, grp, 0)
        svec = (SEQ - 16) + iota16
        v = plsc.load_gather(idx_v, [svec, bcol])
        for t in range(8, 16):
            pltpu.async_copy(
                table_hbm.at[v[t]], rows[p].at[SEQ - 16 + t], gsem[p])

    def wait_gathers(p):
        pltpu.make_async_copy(
            table_hbm.at[pl.ds(0, SEQ)], rows[p], gsem[p]).wait()

    def compute(p):
        rbuf = rows[p]

        def row(j, carry):
            for db in range(D_MODEL // 16):
                sl = pl.ds(db * 16, 16)
                rbuf[j, sl] = rbuf[j, sl] * _SCALE + pe_v[j, sl]
            return carry

        lax.fori_loop(0, SEQ, row, 0, unroll=2)

    def start_out(j, p):
        pltpu.async_copy(
            rows[p], out_hbm.at[pl.ds((wid * B_PER_W + j) * SEQ, SEQ)], osem[p])

    def wait_out(p):
        pltpu.make_async_copy(
            rows[p], out_hbm.at[pl.ds(0, SEQ)], osem[p]).wait()

    enqueue_gathers(0, 0)

    def step(k, carry):
        def for_parity(b, nb):
            @pl.when(k >= 1)
            def _():
                wait_out(nb)

            @pl.when(k + 1 < B_PER_W)
            def _():
                enqueue_gathers(k + 1, nb)

            wait_gathers(b)
            compute(b)
            start_out(k, b)

        @pl.when(k % 2 == 0)
        def _():
            for_parity(0, 1)

        @pl.when(k % 2 == 1)
        def _():
            for_parity(1, 0)

        return carry

    lax.fori_loop(0, B_PER_W, step, 0)
    wait_out((B_PER_W - 1) % 2)


def kernel(x, emb_table):
    scratch = _sc_repack(emb_table.T, emb_table[NFULL * RBLK:])
    out = _sc_gather(x.T, jnp.asarray(_PE), scratch)
    return out.reshape(BATCH, SEQ, D_MODEL)


# speedup vs baseline: 3.5484x; 2.1034x over previous
import functools
import math

import jax
import jax.numpy as jnp
import numpy as np
from jax import lax
from jax.experimental import pallas as pl
from jax.experimental.pallas import tpu as pltpu
from jax.experimental.pallas import tpu_sc as plsc

D_MODEL = 64
VOCAB = 1000000
BATCH = 1024
SEQ = 200

NC = 2
NS = 16
NW = NC * NS

B_PER_W = BATCH // NW


def _make_pe():
    position = np.arange(0, SEQ, dtype=np.float32)[:, None]
    div_term = np.exp(
        np.arange(0, D_MODEL, 2, dtype=np.float32) * -(math.log(10000.0) / D_MODEL)
    )
    pe = np.zeros((SEQ, D_MODEL), dtype=np.float32)
    pe[:, 0::2] = np.sin(position * div_term)
    pe[:, 1::2] = np.cos(position * div_term)
    return pe


_PE = _make_pe()
_SCALE = math.sqrt(D_MODEL)


NGRP = SEQ // 16


@functools.partial(
    pl.kernel,
    mesh=plsc.VectorSubcoreMesh(core_axis_name="c", subcore_axis_name="s"),
    out_type=jax.ShapeDtypeStruct((BATCH * SEQ, D_MODEL), jnp.float32),
    compiler_params=pltpu.CompilerParams(
        use_tc_tiling_on_sc=True, needs_layout_passes=False),
    scratch_types=[
        pltpu.VMEM((SEQ, 128), jnp.int32),
        pltpu.VMEM((SEQ, D_MODEL), jnp.float32),
        pltpu.VMEM((SEQ, D_MODEL), jnp.float32),
        pltpu.VMEM((SEQ, D_MODEL), jnp.float32),
        pltpu.SemaphoreType.DMA,
        pltpu.SemaphoreType.DMA,
        pltpu.SemaphoreType.DMA,
        pltpu.SemaphoreType.DMA,
    ],
)
def _sc_gather(xt_hbm, pe_hbm, table_hbm, out_hbm,
               idx_v, pe_v, rows0, rows1, g0, g1, o0, o1):
    wid = lax.axis_index("s") * NC + lax.axis_index("c")
    rows = (rows0, rows1)
    gsem = (g0, g1)
    osem = (o0, o1)

    blk = (wid // 4) * 128
    coff = (wid % 4) * B_PER_W
    pltpu.sync_copy(xt_hbm.at[:, pl.ds(blk, 128)], idx_v)
    pltpu.sync_copy(pe_hbm, pe_v)

    iota16 = lax.iota(jnp.int32, 16)

    def enqueue_gathers(j, p):
        bcol = iota16 * 0 + (coff + j)

        def grp(g, carry):
            svec = g * 16 + iota16
            v = plsc.load_gather(idx_v, [svec, bcol])
            base = g * 16
            for t in range(16):
                pltpu.async_copy(
                    table_hbm.at[v[t]], rows[p].at[base + t], gsem[p])
            return carry

        lax.fori_loop(0, NGRP, grp, 0)
        svec = (SEQ - 16) + iota16
        v = plsc.load_gather(idx_v, [svec, bcol])
        for t in range(8, 16):
            pltpu.async_copy(
                table_hbm.at[v[t]], rows[p].at[SEQ - 16 + t], gsem[p])

    def wait_gathers(p):
        pltpu.make_async_copy(
            table_hbm.at[pl.ds(0, SEQ)], rows[p], gsem[p]).wait()

    def compute(p):
        rbuf = rows[p]

        @plsc.parallel_loop(0, SEQ, unroll=4)
        def row(j):
            for db in range(D_MODEL // 16):
                sl = pl.ds(db * 16, 16)
                rbuf[j, sl] = rbuf[j, sl] * _SCALE + pe_v[j, sl]

    def start_out(j, p):
        pltpu.async_copy(
            rows[p], out_hbm.at[pl.ds((wid * B_PER_W + j) * SEQ, SEQ)], osem[p])

    def wait_out(p):
        pltpu.make_async_copy(
            rows[p], out_hbm.at[pl.ds(0, SEQ)], osem[p]).wait()

    enqueue_gathers(0, 0)

    def step(k, carry):
        def for_parity(b, nb):
            @pl.when(k >= 1)
            def _():
                wait_out(nb)

            @pl.when(k + 1 < B_PER_W)
            def _():
                enqueue_gathers(k + 1, nb)

            wait_gathers(b)
            compute(b)
            start_out(k, b)

        @pl.when(k % 2 == 0)
        def _():
            for_parity(0, 1)

        @pl.when(k % 2 == 1)
        def _():
            for_parity(1, 0)

        return carry

    lax.fori_loop(0, B_PER_W, step, 0)
    wait_out((B_PER_W - 1) % 2)


def kernel(x, emb_table):
    out = _sc_gather(x.T, jnp.asarray(_PE), emb_table)
    return out.reshape(BATCH, SEQ, D_MODEL)
